# baseline (speedup 1.0000x reference)
import functools

import jax
import jax.numpy as jnp
from jax.experimental import pallas as pl
from jax.experimental.pallas import tpu as pltpu

NODES = 512
INPUT_DIM = 2
HID = 64
OUT_DIM = 1
LAYERS = 2
K = 2
H1 = 13
BATCH = 16
NUM_MAT = 2 * K + 1


def _reorder_w(W, in_size):
    out = W.shape[1]
    return W.reshape(in_size, NUM_MAT, out).transpose(1, 0, 2).reshape(NUM_MAT * in_size, out)


def _dotT(a, b):
    return jax.lax.dot_general(a, b, (((0,), (0,)), ((), ())),
                               preferred_element_type=jnp.float32)


def _dot(a, b):
    return jax.lax.dot_general(a, b, (((1,), (0,)), ((), ())),
                               preferred_element_type=jnp.float32)


def _decoder_kernel(xseq_ref, h0_ref, A_ref,
                    Wg0_ref, bg0_ref, Wc0_ref, bc0_ref,
                    Wg1_ref, bg1_ref, Wc1_ref, bc1_ref,
                    Wfc_ref, bfc_ref, out_ref):
    A = A_ref[...]
    ones_col = jnp.ones((NODES, 1), dtype=jnp.float32)
    ones_row = jnp.ones((1, NODES), dtype=jnp.float32)
    d0 = _dot(A, ones_col)
    d1 = _dot(ones_row, A)
    di0 = jnp.where(d0 > 0.0, 1.0 / d0, 0.0)
    di1 = jnp.where(d1 > 0.0, 1.0 / d1, 0.0)
    A0 = di0 * A
    A1 = A * di1

    A0 = A0.astype(jnp.bfloat16)
    A1 = A1.astype(jnp.bfloat16)

    def s0(x):
        return _dotT(A0, x.astype(jnp.bfloat16))

    def s1(x):
        return _dot(A1, x.astype(jnp.bfloat16))

    def gconv(x0, W, b):
        parts = [x0]
        for S in (s0, s1):
            x1 = S(x0)
            parts.append(x1)
            parts.append(2.0 * S(x1) - x0)
        X = jnp.concatenate(parts, axis=1)
        return _dot(X, W) + b

    h = [h0_ref[0, 0], h0_ref[0, 1]]
    Wg = [Wg0_ref[...], Wg1_ref[...]]
    bg = [bg0_ref[...], bg1_ref[...]]
    Wc = [Wc0_ref[...], Wc1_ref[...]]
    bc = [bc0_ref[...], bc1_ref[...]]
    Wfc = Wfc_ref[...]
    bfc = bfc_ref[...]

    out_ref[0, 0] = jnp.zeros((NODES, OUT_DIM), dtype=jnp.float32)
    hs = []
    for t in range(1, H1):
        x = xseq_ref[0, t - 1]
        for l in range(LAYERS):
            ru = jax.nn.sigmoid(gconv(jnp.concatenate([x, h[l]], axis=1), Wg[l], bg[l]))
            r = ru[:, :HID]
            u = ru[:, HID:]
            c = jnp.tanh(gconv(jnp.concatenate([x, r * h[l]], axis=1), Wc[l], bc[l]))
            h[l] = u * h[l] + (1.0 - u) * c
            x = h[l]
        hs.append(x)
    h_all = jnp.concatenate(hs, axis=0)
    out_ref[0, 1:H1] = (_dot(h_all, Wfc) + bfc).reshape(H1 - 1, NODES, OUT_DIM)


@jax.jit
def _run(xseq, h0, A, Wg0, bg0, Wc0, bc0, Wg1, bg1, Wc1, bc1, Wfc, bfc):
    full = lambda shape: pl.BlockSpec(shape, lambda b: (0,) * len(shape))
    per_b = lambda shape: pl.BlockSpec(shape, lambda b: (b,) + (0,) * (len(shape) - 1))
    out = pl.pallas_call(
        _decoder_kernel,
        grid=(BATCH,),
        in_specs=[
            per_b((1, H1, NODES, INPUT_DIM)),
            per_b((1, LAYERS, NODES, HID)),
            full((NODES, NODES)),
            full(Wg0.shape), full(bg0.shape),
            full(Wc0.shape), full(bc0.shape),
            full(Wg1.shape), full(bg1.shape),
            full(Wc1.shape), full(bc1.shape),
            full(Wfc.shape), full(bfc.shape),
        ],
        out_specs=per_b((1, H1, NODES, OUT_DIM)),
        out_shape=jax.ShapeDtypeStruct((BATCH, H1, NODES, OUT_DIM), jnp.float32),
        compiler_params=pltpu.CompilerParams(dimension_semantics=("parallel",)),
    )(xseq, h0, A, Wg0, bg0, Wc0, bc0, Wg1, bg1, Wc1, bc1, Wfc, bfc)
    return out


def kernel(inputs, init_state, teaching_force_ratio, adj_mx,
           Wg0, bg0, Wc0, bc0, Wg1, bg1, Wc1, bc1, Wfc, bfc):
    del teaching_force_ratio
    xseq = inputs.transpose(1, 0, 2, 3)
    h0 = init_state.reshape(LAYERS, BATCH, NODES, HID).transpose(1, 0, 2, 3)
    Wg0r = _reorder_w(Wg0, INPUT_DIM + HID)
    Wc0r = _reorder_w(Wc0, INPUT_DIM + HID)
    Wg1r = _reorder_w(Wg1, 2 * HID)
    Wc1r = _reorder_w(Wc1, 2 * HID)
    out = _run(xseq, h0, adj_mx, Wg0r, bg0.reshape(1, -1), Wc0r, bc0.reshape(1, -1),
               Wg1r, bg1.reshape(1, -1), Wc1r, bc1.reshape(1, -1),
               Wfc, bfc.reshape(1, -1))
    return out.reshape(BATCH, H1, NODES).transpose(1, 0, 2)


# speedup vs baseline: 1.1165x; 1.1165x over previous
import functools

import jax
import jax.numpy as jnp
from jax.experimental import pallas as pl
from jax.experimental.pallas import tpu as pltpu

NODES = 512
INPUT_DIM = 2
HID = 64
OUT_DIM = 1
LAYERS = 2
K = 2
H1 = 13
BATCH = 16
NUM_MAT = 2 * K + 1
GROUP = 2


def _reorder_w(W, in_size):
    out = W.shape[1]
    return W.reshape(in_size, NUM_MAT, out).transpose(1, 0, 2).reshape(NUM_MAT * in_size, out)


def _dotT(a, b):
    return jax.lax.dot_general(a, b, (((0,), (0,)), ((), ())),
                               preferred_element_type=jnp.float32)


def _dot(a, b):
    return jax.lax.dot_general(a, b, (((1,), (0,)), ((), ())),
                               preferred_element_type=jnp.float32)


def _decoder_kernel(xseq_ref, h0_ref, A_ref,
                    Wg0_ref, bg0_ref, Wc0_ref, bc0_ref,
                    Wg1_ref, bg1_ref, Wc1_ref, bc1_ref,
                    Wfc_ref, bfc_ref, out_ref):
    A = A_ref[...]
    ones_col = jnp.ones((NODES, 1), dtype=jnp.float32)
    ones_row = jnp.ones((1, NODES), dtype=jnp.float32)
    d0 = _dot(A, ones_col)
    d1 = _dot(ones_row, A)
    di0 = jnp.where(d0 > 0.0, 1.0 / d0, 0.0)
    di1 = jnp.where(d1 > 0.0, 1.0 / d1, 0.0)
    A0 = di0 * A
    A1 = A * di1

    def s0(x):
        return _dotT(A0, x)

    def s1(x):
        return _dot(A1, x)

    def gconv(x0, W, b):
        parts = [x0]
        for S in (s0, s1):
            x1 = S(x0)
            parts.append(x1)
            parts.append(2.0 * S(x1) - x0)
        X = jnp.concatenate(parts, axis=1)
        return _dot(X, W) + b

    Wg = [Wg0_ref[...], Wg1_ref[...]]
    bg = [bg0_ref[...], bg1_ref[...]]
    Wc = [Wc0_ref[...], Wc1_ref[...]]
    bc = [bc0_ref[...], bc1_ref[...]]
    Wfc = Wfc_ref[...]
    bfc = bfc_ref[...]

    h = [[h0_ref[g, l] for l in range(LAYERS)] for g in range(GROUP)]
    hs = []
    for g in range(GROUP):
        out_ref[g, 0] = jnp.zeros((NODES, OUT_DIM), dtype=jnp.float32)
    for t in range(1, H1):
        xg = [xseq_ref[g, t - 1] for g in range(GROUP)]
        for l in range(LAYERS):
            for g in range(GROUP):
                x = xg[g]
                ru = jax.nn.sigmoid(gconv(jnp.concatenate([x, h[g][l]], axis=1), Wg[l], bg[l]))
                r = ru[:, :HID]
                u = ru[:, HID:]
                c = jnp.tanh(gconv(jnp.concatenate([x, r * h[g][l]], axis=1), Wc[l], bc[l]))
                h[g][l] = u * h[g][l] + (1.0 - u) * c
                xg[g] = h[g][l]
        hs.append(xg)
    for g in range(GROUP):
        h_all = jnp.concatenate([hstep[g] for hstep in hs], axis=0)
        out_ref[g, 1:H1] = (_dot(h_all, Wfc) + bfc).reshape(H1 - 1, NODES, OUT_DIM)


@jax.jit
def _run(xseq, h0, A, Wg0, bg0, Wc0, bc0, Wg1, bg1, Wc1, bc1, Wfc, bfc):
    full = lambda shape: pl.BlockSpec(shape, lambda b: (0,) * len(shape))
    per_b = lambda shape: pl.BlockSpec(shape, lambda b: (b,) + (0,) * (len(shape) - 1))
    out = pl.pallas_call(
        _decoder_kernel,
        grid=(BATCH // GROUP,),
        in_specs=[
            per_b((GROUP, H1, NODES, INPUT_DIM)),
            per_b((GROUP, LAYERS, NODES, HID)),
            full((NODES, NODES)),
            full(Wg0.shape), full(bg0.shape),
            full(Wc0.shape), full(bc0.shape),
            full(Wg1.shape), full(bg1.shape),
            full(Wc1.shape), full(bc1.shape),
            full(Wfc.shape), full(bfc.shape),
        ],
        out_specs=per_b((GROUP, H1, NODES, OUT_DIM)),
        out_shape=jax.ShapeDtypeStruct((BATCH, H1, NODES, OUT_DIM), jnp.float32),
        compiler_params=pltpu.CompilerParams(dimension_semantics=("parallel",)),
    )(xseq, h0, A, Wg0, bg0, Wc0, bc0, Wg1, bg1, Wc1, bc1, Wfc, bfc)
    return out


def kernel(inputs, init_state, teaching_force_ratio, adj_mx,
           Wg0, bg0, Wc0, bc0, Wg1, bg1, Wc1, bc1, Wfc, bfc):
    del teaching_force_ratio
    xseq = inputs.transpose(1, 0, 2, 3)
    h0 = init_state.reshape(LAYERS, BATCH, NODES, HID).transpose(1, 0, 2, 3)
    Wg0r = _reorder_w(Wg0, INPUT_DIM + HID)
    Wc0r = _reorder_w(Wc0, INPUT_DIM + HID)
    Wg1r = _reorder_w(Wg1, 2 * HID)
    Wc1r = _reorder_w(Wc1, 2 * HID)
    out = _run(xseq, h0, adj_mx, Wg0r, bg0.reshape(1, -1), Wc0r, bc0.reshape(1, -1),
               Wg1r, bg1.reshape(1, -1), Wc1r, bc1.reshape(1, -1),
               Wfc, bfc.reshape(1, -1))
    return out.reshape(BATCH, H1, NODES).transpose(1, 0, 2)


# speedup vs baseline: 1.1954x; 1.0707x over previous
import functools

import jax
import jax.numpy as jnp
from jax.experimental import pallas as pl
from jax.experimental.pallas import tpu as pltpu

NODES = 512
INPUT_DIM = 2
HID = 64
OUT_DIM = 1
LAYERS = 2
K = 2
H1 = 13
BATCH = 16
NUM_MAT = 2 * K + 1
GROUP = 4


def _reorder_w(W, in_size):
    out = W.shape[1]
    return W.reshape(in_size, NUM_MAT, out).transpose(1, 0, 2).reshape(NUM_MAT * in_size, out)


def _dotT(a, b):
    return jax.lax.dot_general(a, b, (((0,), (0,)), ((), ())),
                               preferred_element_type=jnp.float32)


def _dot(a, b):
    return jax.lax.dot_general(a, b, (((1,), (0,)), ((), ())),
                               preferred_element_type=jnp.float32)


def _decoder_kernel(xseq_ref, h0_ref, A_ref,
                    Wg0_ref, bg0_ref, Wc0_ref, bc0_ref,
                    Wg1_ref, bg1_ref, Wc1_ref, bc1_ref,
                    Wfc_ref, bfc_ref, out_ref):
    A = A_ref[...]
    ones_col = jnp.ones((NODES, 1), dtype=jnp.float32)
    ones_row = jnp.ones((1, NODES), dtype=jnp.float32)
    d0 = _dot(A, ones_col)
    d1 = _dot(ones_row, A)
    di0 = jnp.where(d0 > 0.0, 1.0 / d0, 0.0)
    di1 = jnp.where(d1 > 0.0, 1.0 / d1, 0.0)
    A0 = di0 * A
    A1 = A * di1

    def s0(x):
        return _dotT(A0, x)

    def s1(x):
        return _dot(A1, x)

    def gconv(x0, W, b):
        parts = [x0]
        for S in (s0, s1):
            x1 = S(x0)
            parts.append(x1)
            parts.append(2.0 * S(x1) - x0)
        X = jnp.concatenate(parts, axis=1)
        return _dot(X, W) + b

    Wg = [Wg0_ref[...], Wg1_ref[...]]
    bg = [bg0_ref[...], bg1_ref[...]]
    Wc = [Wc0_ref[...], Wc1_ref[...]]
    bc = [bc0_ref[...], bc1_ref[...]]
    Wfc = Wfc_ref[...]
    bfc = bfc_ref[...]

    h = [[h0_ref[g, l] for l in range(LAYERS)] for g in range(GROUP)]
    hs = []
    for g in range(GROUP):
        out_ref[g, 0] = jnp.zeros((NODES,), dtype=jnp.float32)
    for t in range(1, H1):
        xg = [xseq_ref[g, t - 1] for g in range(GROUP)]
        for l in range(LAYERS):
            for g in range(GROUP):
                x = xg[g]
                ru = jax.nn.sigmoid(gconv(jnp.concatenate([x, h[g][l]], axis=1), Wg[l], bg[l]))
                r = ru[:, :HID]
                u = ru[:, HID:]
                c = jnp.tanh(gconv(jnp.concatenate([x, r * h[g][l]], axis=1), Wc[l], bc[l]))
                h[g][l] = u * h[g][l] + (1.0 - u) * c
                xg[g] = h[g][l]
        hs.append(xg)
    for g in range(GROUP):
        h_all = jnp.concatenate([hstep[g] for hstep in hs], axis=0)
        p = jax.lax.dot_general(Wfc, h_all, (((0,), (1,)), ((), ())),
                                preferred_element_type=jnp.float32) + bfc[0, 0]
        for t in range(1, H1):
            out_ref[g, t] = p[0, (t - 1) * NODES:t * NODES]


@jax.jit
def _run(xseq, h0, A, Wg0, bg0, Wc0, bc0, Wg1, bg1, Wc1, bc1, Wfc, bfc):
    full = lambda shape: pl.BlockSpec(shape, lambda b: (0,) * len(shape))
    per_b = lambda shape: pl.BlockSpec(shape, lambda b: (b,) + (0,) * (len(shape) - 1))
    out = pl.pallas_call(
        _decoder_kernel,
        grid=(BATCH // GROUP,),
        in_specs=[
            per_b((GROUP, H1, NODES, INPUT_DIM)),
            per_b((GROUP, LAYERS, NODES, HID)),
            full((NODES, NODES)),
            full(Wg0.shape), full(bg0.shape),
            full(Wc0.shape), full(bc0.shape),
            full(Wg1.shape), full(bg1.shape),
            full(Wc1.shape), full(bc1.shape),
            full(Wfc.shape), full(bfc.shape),
        ],
        out_specs=per_b((GROUP, H1, NODES)),
        out_shape=jax.ShapeDtypeStruct((BATCH, H1, NODES), jnp.float32),
        compiler_params=pltpu.CompilerParams(dimension_semantics=("parallel",),
                                             vmem_limit_bytes=100 * 1024 * 1024),
    )(xseq, h0, A, Wg0, bg0, Wc0, bc0, Wg1, bg1, Wc1, bc1, Wfc, bfc)
    return out


def kernel(inputs, init_state, teaching_force_ratio, adj_mx,
           Wg0, bg0, Wc0, bc0, Wg1, bg1, Wc1, bc1, Wfc, bfc):
    del teaching_force_ratio
    xseq = inputs.transpose(1, 0, 2, 3)
    h0 = init_state.reshape(LAYERS, BATCH, NODES, HID).transpose(1, 0, 2, 3)
    Wg0r = _reorder_w(Wg0, INPUT_DIM + HID)
    Wc0r = _reorder_w(Wc0, INPUT_DIM + HID)
    Wg1r = _reorder_w(Wg1, 2 * HID)
    Wc1r = _reorder_w(Wc1, 2 * HID)
    out = _run(xseq, h0, adj_mx, Wg0r, bg0.reshape(1, -1), Wc0r, bc0.reshape(1, -1),
               Wg1r, bg1.reshape(1, -1), Wc1r, bc1.reshape(1, -1),
               Wfc, bfc.reshape(1, -1))
    return out.transpose(1, 0, 2)


# speedup vs baseline: 1.2189x; 1.0196x over previous
import functools

import jax
import jax.numpy as jnp
from jax.experimental import pallas as pl
from jax.experimental.pallas import tpu as pltpu

NODES = 512
INPUT_DIM = 2
HID = 64
OUT_DIM = 1
LAYERS = 2
K = 2
H1 = 13
BATCH = 16
NUM_MAT = 2 * K + 1
GROUP = 4


def _reorder_w(W, in_size):
    out = W.shape[1]
    return W.reshape(in_size, NUM_MAT, out).transpose(1, 0, 2).reshape(NUM_MAT * in_size, out)


def _dotT(a, b):
    return jax.lax.dot_general(a, b, (((0,), (0,)), ((), ())),
                               preferred_element_type=jnp.float32)


def _dot(a, b):
    return jax.lax.dot_general(a, b, (((1,), (0,)), ((), ())),
                               preferred_element_type=jnp.float32)


def _decoder_kernel(xseq_ref, h0_ref, A_ref,
                    Wg0_ref, bg0_ref, Wc0_ref, bc0_ref,
                    Wg1_ref, bg1_ref, Wc1_ref, bc1_ref,
                    Wfc_ref, bfc_ref, out_ref):
    A = A_ref[...]
    ones_col = jnp.ones((NODES, 1), dtype=jnp.float32)
    ones_row = jnp.ones((1, NODES), dtype=jnp.float32)
    d0 = _dot(A, ones_col)
    d1 = _dot(ones_row, A)
    di0 = jnp.where(d0 > 0.0, 1.0 / d0, 0.0)
    di1 = jnp.where(d1 > 0.0, 1.0 / d1, 0.0)
    A0 = di0 * A
    A1 = A * di1

    def s0(x):
        return _dotT(A0, x)

    def s1(x):
        return _dot(A1, x)

    def gconv(x0, W, b):
        parts = [x0]
        for S in (s0, s1):
            x1 = S(x0)
            parts.append(x1)
            parts.append(2.0 * S(x1) - x0)
        X = jnp.concatenate(parts, axis=1)
        return _dot(X, W) + b

    Wg = [Wg0_ref[...], Wg1_ref[...]]
    bg = [bg0_ref[...], bg1_ref[...]]
    Wc = [Wc0_ref[...], Wc1_ref[...]]
    bc = [bc0_ref[...], bc1_ref[...]]
    Wfc = Wfc_ref[...]
    bfc = bfc_ref[...]

    h = [[h0_ref[g, l] for l in range(LAYERS)] for g in range(GROUP)]
    hs = []
    for g in range(GROUP):
        out_ref[g, 0] = jnp.zeros((NODES,), dtype=jnp.float32)
    for t in range(1, H1):
        xg = [xseq_ref[g, :, :, t - 1] for g in range(GROUP)]
        for l in range(LAYERS):
            for g in range(GROUP):
                x = xg[g]
                ru = jax.nn.sigmoid(gconv(jnp.concatenate([x, h[g][l]], axis=1), Wg[l], bg[l]))
                r = ru[:, :HID]
                u = ru[:, HID:]
                c = jnp.tanh(gconv(jnp.concatenate([x, r * h[g][l]], axis=1), Wc[l], bc[l]))
                h[g][l] = u * h[g][l] + (1.0 - u) * c
                xg[g] = h[g][l]
        hs.append(xg)
    for g in range(GROUP):
        h_all = jnp.concatenate([hstep[g] for hstep in hs], axis=0)
        p = jax.lax.dot_general(Wfc, h_all, (((0,), (1,)), ((), ())),
                                preferred_element_type=jnp.float32) + bfc[0, 0]
        for t in range(1, H1):
            out_ref[g, t] = p[0, (t - 1) * NODES:t * NODES]


@jax.jit
def _run(xseq, h0, A, Wg0, bg0, Wc0, bc0, Wg1, bg1, Wc1, bc1, Wfc, bfc):
    full = lambda shape: pl.BlockSpec(shape, lambda b: (0,) * len(shape))
    per_b = lambda shape: pl.BlockSpec(shape, lambda b: (b,) + (0,) * (len(shape) - 1))
    out = pl.pallas_call(
        _decoder_kernel,
        grid=(BATCH // GROUP,),
        in_specs=[
            per_b((GROUP, NODES, INPUT_DIM, H1)),
            per_b((GROUP, LAYERS, NODES, HID)),
            full((NODES, NODES)),
            full(Wg0.shape), full(bg0.shape),
            full(Wc0.shape), full(bc0.shape),
            full(Wg1.shape), full(bg1.shape),
            full(Wc1.shape), full(bc1.shape),
            full(Wfc.shape), full(bfc.shape),
        ],
        out_specs=per_b((GROUP, H1, NODES)),
        out_shape=jax.ShapeDtypeStruct((BATCH, H1, NODES), jnp.float32),
        compiler_params=pltpu.CompilerParams(dimension_semantics=("parallel",),
                                             vmem_limit_bytes=100 * 1024 * 1024),
    )(xseq, h0, A, Wg0, bg0, Wc0, bc0, Wg1, bg1, Wc1, bc1, Wfc, bfc)
    return out


def kernel(inputs, init_state, teaching_force_ratio, adj_mx,
           Wg0, bg0, Wc0, bc0, Wg1, bg1, Wc1, bc1, Wfc, bfc):
    del teaching_force_ratio
    xseq = inputs.transpose(1, 2, 3, 0)
    h0 = init_state.reshape(LAYERS, BATCH, NODES, HID).transpose(1, 0, 2, 3)
    Wg0r = _reorder_w(Wg0, INPUT_DIM + HID)
    Wc0r = _reorder_w(Wc0, INPUT_DIM + HID)
    Wg1r = _reorder_w(Wg1, 2 * HID)
    Wc1r = _reorder_w(Wc1, 2 * HID)
    out = _run(xseq, h0, adj_mx, Wg0r, bg0.reshape(1, -1), Wc0r, bc0.reshape(1, -1),
               Wg1r, bg1.reshape(1, -1), Wc1r, bc1.reshape(1, -1),
               Wfc, bfc.reshape(1, -1))
    return out.transpose(1, 0, 2)


# speedup vs baseline: 1.2424x; 1.0193x over previous
import functools

import jax
import jax.numpy as jnp
from jax.experimental import pallas as pl
from jax.experimental.pallas import tpu as pltpu

NODES = 512
INPUT_DIM = 2
HID = 64
OUT_DIM = 1
LAYERS = 2
K = 2
H1 = 13
BATCH = 16
NUM_MAT = 2 * K + 1
GROUP = 8


def _reorder_w(W, in_size):
    out = W.shape[1]
    return W.reshape(in_size, NUM_MAT, out).transpose(1, 0, 2).reshape(NUM_MAT * in_size, out)


def _dotT(a, b):
    return jax.lax.dot_general(a, b, (((0,), (0,)), ((), ())),
                               preferred_element_type=jnp.float32)


def _dot(a, b):
    return jax.lax.dot_general(a, b, (((1,), (0,)), ((), ())),
                               preferred_element_type=jnp.float32)


def _decoder_kernel(xseq_ref, h0_ref, A_ref,
                    Wg0_ref, bg0_ref, Wc0_ref, bc0_ref,
                    Wg1_ref, bg1_ref, Wc1_ref, bc1_ref,
                    Wfc_ref, bfc_ref, out_ref):
    A = A_ref[...]
    ones_col = jnp.ones((NODES, 1), dtype=jnp.float32)
    ones_row = jnp.ones((1, NODES), dtype=jnp.float32)
    d0 = _dot(A, ones_col)
    d1 = _dot(ones_row, A)
    di0 = jnp.where(d0 > 0.0, 1.0 / d0, 0.0)
    di1 = jnp.where(d1 > 0.0, 1.0 / d1, 0.0)
    A0 = di0 * A
    A1 = A * di1

    def s0(x):
        return _dotT(A0, x)

    def s1(x):
        return _dot(A1, x)

    def gconv(x0, W, b):
        parts = [x0]
        for S in (s0, s1):
            x1 = S(x0)
            parts.append(x1)
            parts.append(2.0 * S(x1) - x0)
        X = jnp.concatenate(parts, axis=1)
        return _dot(X, W) + b

    Wg = [Wg0_ref[...], Wg1_ref[...]]
    bg = [bg0_ref[...], bg1_ref[...]]
    Wc = [Wc0_ref[...], Wc1_ref[...]]
    bc = [bc0_ref[...], bc1_ref[...]]
    Wfc = Wfc_ref[...]
    bfc = bfc_ref[...]

    h = [[h0_ref[g, l] for l in range(LAYERS)] for g in range(GROUP)]
    hs = []
    for g in range(GROUP):
        out_ref[g, 0] = jnp.zeros((NODES,), dtype=jnp.float32)
    for t in range(1, H1):
        xg = [xseq_ref[g, :, :, t - 1] for g in range(GROUP)]
        for l in range(LAYERS):
            for g in range(GROUP):
                x = xg[g]
                ru = jax.nn.sigmoid(gconv(jnp.concatenate([x, h[g][l]], axis=1), Wg[l], bg[l]))
                r = ru[:, :HID]
                u = ru[:, HID:]
                c = jnp.tanh(gconv(jnp.concatenate([x, r * h[g][l]], axis=1), Wc[l], bc[l]))
                h[g][l] = u * h[g][l] + (1.0 - u) * c
                xg[g] = h[g][l]
        hs.append(xg)
    for g in range(GROUP):
        h_all = jnp.concatenate([hstep[g] for hstep in hs], axis=0)
        p = jax.lax.dot_general(Wfc, h_all, (((0,), (1,)), ((), ())),
                                preferred_element_type=jnp.float32) + bfc[0, 0]
        for t in range(1, H1):
            out_ref[g, t] = p[0, (t - 1) * NODES:t * NODES]


@jax.jit
def _run(xseq, h0, A, Wg0, bg0, Wc0, bc0, Wg1, bg1, Wc1, bc1, Wfc, bfc):
    full = lambda shape: pl.BlockSpec(shape, lambda b: (0,) * len(shape))
    per_b = lambda shape: pl.BlockSpec(shape, lambda b: (b,) + (0,) * (len(shape) - 1))
    out = pl.pallas_call(
        _decoder_kernel,
        grid=(BATCH // GROUP,),
        in_specs=[
            per_b((GROUP, NODES, INPUT_DIM, H1)),
            per_b((GROUP, LAYERS, NODES, HID)),
            full((NODES, NODES)),
            full(Wg0.shape), full(bg0.shape),
            full(Wc0.shape), full(bc0.shape),
            full(Wg1.shape), full(bg1.shape),
            full(Wc1.shape), full(bc1.shape),
            full(Wfc.shape), full(bfc.shape),
        ],
        out_specs=per_b((GROUP, H1, NODES)),
        out_shape=jax.ShapeDtypeStruct((BATCH, H1, NODES), jnp.float32),
        compiler_params=pltpu.CompilerParams(dimension_semantics=("parallel",),
                                             vmem_limit_bytes=100 * 1024 * 1024),
    )(xseq, h0, A, Wg0, bg0, Wc0, bc0, Wg1, bg1, Wc1, bc1, Wfc, bfc)
    return out


def kernel(inputs, init_state, teaching_force_ratio, adj_mx,
           Wg0, bg0, Wc0, bc0, Wg1, bg1, Wc1, bc1, Wfc, bfc):
    del teaching_force_ratio
    xseq = inputs.transpose(1, 2, 3, 0)
    h0 = init_state.reshape(LAYERS, BATCH, NODES, HID).transpose(1, 0, 2, 3)
    Wg0r = _reorder_w(Wg0, INPUT_DIM + HID)
    Wc0r = _reorder_w(Wc0, INPUT_DIM + HID)
    Wg1r = _reorder_w(Wg1, 2 * HID)
    Wc1r = _reorder_w(Wc1, 2 * HID)
    out = _run(xseq, h0, adj_mx, Wg0r, bg0.reshape(1, -1), Wc0r, bc0.reshape(1, -1),
               Wg1r, bg1.reshape(1, -1), Wc1r, bc1.reshape(1, -1),
               Wfc, bfc.reshape(1, -1))
    return out.transpose(1, 0, 2)
